# baseline (speedup 1.0000x reference)
import functools

import jax
import jax.numpy as jnp
from jax import lax
from jax.experimental import pallas as pl
from jax.experimental.pallas import tpu as pltpu
from jax.experimental.pallas import tpu_sc as plsc

NC = 2
NS = 16
NW = NC * NS

BATCH = 16384
HIST = 50
VECDIM = 32
B = BATCH * HIST
BPW = B // NW
BBLK = 128
NBT = BATCH // BBLK
BT_PER_W = NBT // NW
NBLOCK = BT_PER_W * HIST

_mesh = plsc.VectorSubcoreMesh(
    core_axis_name="c", subcore_axis_name="s", num_cores=NC, num_subcores=NS
)

_IOTA16 = tuple(range(16))

WORDNUM = 1000000
WORDS = WORDNUM * VECDIM
NTILE_BULK = WORDNUM // 128
TAIL = WORDNUM - 128 * NTILE_BULK
TRIPS = 250
MAXSTART = NTILE_BULK - TRIPS


@functools.partial(
    pl.kernel,
    out_type=jax.ShapeDtypeStruct((WORDS // 128, 128), jnp.float32),
    mesh=_mesh,
    scratch_types=[
        pltpu.VMEM((VECDIM // 8, 8, 129), jnp.float32),
        pltpu.VMEM((VECDIM // 8, 8, 129), jnp.float32),
        pltpu.VMEM((VECDIM // 8, 8, 129), jnp.float32),
        pltpu.VMEM((VECDIM // 8, 8, 129), jnp.float32),
        pltpu.VMEM((32, 128), jnp.float32),
        pltpu.VMEM((32, 128), jnp.float32),
        pltpu.VMEM((32, 128), jnp.float32),
        pltpu.VMEM((32, 128), jnp.float32),
        pltpu.SemaphoreType.DMA,
        pltpu.SemaphoreType.DMA,
        pltpu.SemaphoreType.DMA,
        pltpu.SemaphoreType.DMA,
        pltpu.SemaphoreType.DMA,
        pltpu.SemaphoreType.DMA,
        pltpu.SemaphoreType.DMA,
        pltpu.SemaphoreType.DMA,
    ],
    compiler_params=pltpu.CompilerParams(
        use_tc_tiling_on_sc=True,
        needs_layout_passes=False,
        disable_bounds_checks=True,
    ),
)
def _detile_kernel(tabt_hbm, out_hbm, tb0, tb1, tb2, tb3, db0, db1, db2, db3,
                   isem0, isem1, isem2, isem3, osem0, osem1, osem2, osem3):
    wid = lax.axis_index("s") * NC + lax.axis_index("c")
    start = jnp.minimum(244 * wid, MAXSTART)

    tb = (tb0, tb1, tb2, tb3)
    db = (db0, db1, db2, db3)
    isem = (isem0, isem1, isem2, isem3)
    osem = (osem0, osem1, osem2, osem3)

    iota = jnp.arange(16, dtype=jnp.int32)
    zero16 = jnp.zeros((16,), dtype=jnp.int32)

    def in_copies(wt, buf):
        return [
            pltpu.make_async_copy(
                tabt_hbm.at[pl.ds(c, 1), pl.ds(128 * wt, 128)],
                tb[buf].at[c // 8, pl.ds(c % 8, 1), pl.ds(0, 128)],
                isem[buf],
            )
            for c in range(VECDIM)
        ]

    def start_in(wt, buf):
        for c in in_copies(wt, buf):
            c.start()

    def wait_in(wt, buf):
        for c in in_copies(wt, buf):
            c.wait()

    def transpose(buf):
        @plsc.parallel_loop(0, 128, step=1, unroll=4)
        def _l(l):
            lvec = zero16 + l
            for c0 in (0, 16):
                ct_vec = (c0 + iota) // 8
                s_vec = (c0 + iota) % 8
                vec = plsc.load_gather(tb[buf], [ct_vec, s_vec, lvec])
                w = l * VECDIM + c0
                db[buf][w // 128, pl.ds(w % 128, 16)] = vec

    def out_copy(wt, buf):
        return pltpu.make_async_copy(
            db[buf], out_hbm.at[pl.ds(32 * wt, 32), :], osem[buf]
        )

    def complete(j, buf):
        wait_in(start + j, buf)
        pl.when(j >= 4)(lambda: out_copy(start + j - 4, buf).wait())
        transpose(buf)
        out_copy(start + j, buf).start()

    start_in(start, 0)
    start_in(start + 1, 1)
    start_in(start + 2, 2)

    @pl.loop(3, TRIPS - 3, step=4)
    def _steady(t):
        for d in range(4):
            start_in(start + t + d, (3 + d) % 4)
            complete(t + d - 3, d % 4)

    start_in(start + 247, 3)
    complete(244, 0)
    start_in(start + 248, 0)
    complete(245, 1)
    start_in(start + 249, 1)
    complete(246, 2)
    complete(247, 3)
    complete(248, 0)
    complete(249, 1)
    out_copy(start + 246, 2).wait()
    out_copy(start + 247, 3).wait()
    out_copy(start + 248, 0).wait()
    out_copy(start + 249, 1).wait()


@functools.partial(
    pl.kernel,
    out_type=jax.ShapeDtypeStruct((HIST, VECDIM // 8, NBT, 8, BBLK), jnp.float32),
    mesh=_mesh,
    scratch_types=[
        pltpu.VMEM((TAIL, VECDIM), jnp.float32),
        pltpu.VMEM((BPW,), jnp.int32),
        pltpu.VMEM((BBLK,), jnp.int32),
        pltpu.VMEM((BBLK,), jnp.int32),
        pltpu.VMEM((BBLK, VECDIM), jnp.float32),
        pltpu.VMEM((BBLK, VECDIM), jnp.float32),
        pltpu.VMEM((VECDIM // 8, 8, BBLK + 1), jnp.float32),
        pltpu.VMEM((VECDIM // 8, 8, BBLK + 1), jnp.float32),
        pltpu.SemaphoreType.DMA,
        pltpu.SemaphoreType.DMA,
        pltpu.SemaphoreType.DMA,
        pltpu.SemaphoreType.DMA,
    ],
    compiler_params=pltpu.CompilerParams(
        use_tc_tiling_on_sc=False,
        needs_layout_passes=False,
        disable_bounds_checks=True,
    ),
)
def _gather_kernel(idx_hbm, table_flat_hbm, tail_hbm, out_hbm, tail_v, idx_v,
                   ih0, ih1, r0, r1, o0, o1, gsem0, gsem1, wsem0, wsem1):
    wid = lax.axis_index("s") * NC + lax.axis_index("c")
    base = wid * BPW
    table_hbm = table_flat_hbm
    LIMIT = 128 * NTILE_BULK

    ih = (ih0, ih1)
    rows = (r0, r1)
    oblk = (o0, o1)
    gsem = (gsem0, gsem1)
    wsem = (wsem0, wsem1)

    iota = jnp.arange(16, dtype=jnp.int32)

    pltpu.sync_copy(idx_hbm.at[pl.ds(base, BPW)], idx_v)
    pltpu.sync_copy(tail_hbm, tail_v)

    def block_params(t):
        bt_i = t // HIST
        h = t - bt_i * HIST
        return bt_i, h

    def repack_and_gather(t, buf):
        bt_i, h = block_params(t)
        pos0 = bt_i * (HIST * BBLK) + h
        for k in range(8):
            vec = plsc.load_gather(idx_v, [iota * HIST + (pos0 + 800 * k)])
            ih[buf][pl.ds(16 * k, 16)] = vec
        return pltpu.async_copy(table_hbm.at[ih[buf]], rows[buf], gsem[buf])

    def wait_gather(buf):
        pltpu.make_async_copy(table_hbm.at[ih[buf]], rows[buf], gsem[buf]).wait()

    def transpose_and_write(t, buf):
        bt_i, h = block_params(t)

        @plsc.parallel_loop(0, BBLK, step=1, unroll=4)
        def _l(l):
            lvec = jnp.full((16,), 0, dtype=jnp.int32) + l
            for c0 in (0, 16):
                vec = rows[buf][l, pl.ds(c0, 16)]
                ct_vec = (c0 + iota) // 8
                s_vec = (c0 + iota) % 8
                plsc.store_scatter(oblk[buf], [ct_vec, s_vec, lvec], vec)

        acc = jnp.zeros((16,), dtype=jnp.int32)
        for m in range(8):
            tl = ih[buf][pl.ds(16 * m, 16)]
            acc = acc + (tl >= LIMIT).astype(jnp.int32)
        any_tail = lax.reduce_max(acc, (0,))

        @pl.when(any_tail > 0)
        def _fixup():
            for m in range(8):
                tl = ih[buf][pl.ds(16 * m, 16)]
                mask = tl >= LIMIT
                tv = jnp.maximum(tl - LIMIT, 0)
                lv = iota + 16 * m
                for c in range(VECDIM):
                    cv = jnp.full((16,), c, dtype=jnp.int32)
                    vals = plsc.load_gather(tail_v, [tv, cv], mask=mask)
                    plsc.store_scatter(
                        oblk[buf],
                        [jnp.full((16,), c // 8, dtype=jnp.int32),
                         jnp.full((16,), c % 8, dtype=jnp.int32), lv],
                        vals, mask=mask,
                    )

        dst = out_hbm.at[h, :, wid * BT_PER_W + bt_i]
        src = oblk[buf].at[:, :, pl.ds(0, BBLK)]
        return pltpu.async_copy(src, dst, wsem[buf])

    def wait_write(t, buf):
        bt_i, h = block_params(t)
        dst = out_hbm.at[h, :, wid * BT_PER_W + bt_i]
        src = oblk[buf].at[:, :, pl.ds(0, BBLK)]
        pltpu.make_async_copy(src, dst, wsem[buf]).wait()

    def half(t, tbuf):
        repack_and_gather(t, tbuf)
        prev = t - 1
        pbuf = 1 - tbuf
        wait_gather(pbuf)
        pl.when(t >= 3)(lambda: wait_write(prev - 2, pbuf))
        transpose_and_write(prev, pbuf)

    repack_and_gather(0, 0)

    @pl.loop(1, NBLOCK - 1, step=2)
    def _steady(t):
        half(t, 1)
        half(t + 1, 0)

    t_last = NBLOCK - 1
    repack_and_gather(t_last, 1)
    wait_gather(0)
    wait_write(t_last - 3, 0)
    transpose_and_write(t_last - 1, 0)
    wait_gather(1)
    wait_write(t_last - 2, 1)
    transpose_and_write(t_last, 1)
    wait_write(t_last - 1, 0)
    wait_write(t_last, 1)


def kernel(x, table):
    idx = x.reshape(B).astype(jnp.int32)
    table_dense = _detile_kernel(table.T).reshape(WORDNUM, VECDIM)
    tail = lax.slice(table, (128 * NTILE_BULK, 0), (WORDNUM, VECDIM))
    out5d = _gather_kernel(idx, table_dense, tail)
    return out5d.transpose(2, 4, 0, 1, 3).reshape(BATCH, HIST, VECDIM)


# speedup vs baseline: 1.0329x; 1.0329x over previous
import functools

import jax
import jax.numpy as jnp
from jax import lax
from jax.experimental import pallas as pl
from jax.experimental.pallas import tpu as pltpu
from jax.experimental.pallas import tpu_sc as plsc

NC = 2
NS = 16
NW = NC * NS

BATCH = 16384
HIST = 50
VECDIM = 32
B = BATCH * HIST
BPW = B // NW
BBLK = 128
NBT = BATCH // BBLK
BT_PER_W = NBT // NW
NBLOCK = BT_PER_W * HIST

_mesh = plsc.VectorSubcoreMesh(
    core_axis_name="c", subcore_axis_name="s", num_cores=NC, num_subcores=NS
)

_IOTA16 = tuple(range(16))

WORDNUM = 1000000
WORDS = WORDNUM * VECDIM
NTILE_BULK = WORDNUM // 128
TAIL = WORDNUM - 128 * NTILE_BULK
TRIPS = 250
MAXSTART = NTILE_BULK - TRIPS


@functools.partial(
    pl.kernel,
    out_type=jax.ShapeDtypeStruct((WORDS // 128, 128), jnp.float32),
    mesh=_mesh,
    scratch_types=[
        pltpu.VMEM((VECDIM // 8, 8, 129), jnp.float32),
        pltpu.VMEM((VECDIM // 8, 8, 129), jnp.float32),
        pltpu.VMEM((VECDIM // 8, 8, 129), jnp.float32),
        pltpu.VMEM((VECDIM // 8, 8, 129), jnp.float32),
        pltpu.VMEM((32, 128), jnp.float32),
        pltpu.VMEM((32, 128), jnp.float32),
        pltpu.VMEM((32, 128), jnp.float32),
        pltpu.VMEM((32, 128), jnp.float32),
        pltpu.SemaphoreType.DMA,
        pltpu.SemaphoreType.DMA,
        pltpu.SemaphoreType.DMA,
        pltpu.SemaphoreType.DMA,
        pltpu.SemaphoreType.DMA,
        pltpu.SemaphoreType.DMA,
        pltpu.SemaphoreType.DMA,
        pltpu.SemaphoreType.DMA,
    ],
    compiler_params=pltpu.CompilerParams(
        use_tc_tiling_on_sc=True,
        needs_layout_passes=False,
        disable_bounds_checks=True,
    ),
)
def _detile_kernel(tabt_hbm, out_hbm, tb0, tb1, tb2, tb3, db0, db1, db2, db3,
                   isem0, isem1, isem2, isem3, osem0, osem1, osem2, osem3):
    wid = lax.axis_index("s") * NC + lax.axis_index("c")
    start = jnp.minimum(244 * wid, MAXSTART)

    tb = (tb0, tb1, tb2, tb3)
    db = (db0, db1, db2, db3)
    isem = (isem0, isem1, isem2, isem3)
    osem = (osem0, osem1, osem2, osem3)

    iota = jnp.arange(16, dtype=jnp.int32)
    zero16 = jnp.zeros((16,), dtype=jnp.int32)

    def in_copies(wt, buf):
        return [
            pltpu.make_async_copy(
                tabt_hbm.at[pl.ds(c, 1), pl.ds(128 * wt, 128)],
                tb[buf].at[c // 8, pl.ds(c % 8, 1), pl.ds(0, 128)],
                isem[buf],
            )
            for c in range(VECDIM)
        ]

    def start_in(wt, buf):
        for c in in_copies(wt, buf):
            c.start()

    def wait_in(wt, buf):
        for ct in range(VECDIM // 8):
            pltpu.make_async_copy(
                tabt_hbm.at[pl.ds(8 * ct, 8), pl.ds(128 * wt, 128)],
                tb[buf].at[ct, :, pl.ds(0, 128)],
                isem[buf],
            ).wait()

    def transpose(buf):
        @plsc.parallel_loop(0, 128, step=1, unroll=8)
        def _l(l):
            lvec = zero16 + l
            for c0 in (0, 16):
                ct_vec = (c0 + iota) // 8
                s_vec = (c0 + iota) % 8
                vec = plsc.load_gather(tb[buf], [ct_vec, s_vec, lvec])
                w = l * VECDIM + c0
                db[buf][w // 128, pl.ds(w % 128, 16)] = vec

    def out_copy(wt, buf):
        return pltpu.make_async_copy(
            db[buf], out_hbm.at[pl.ds(32 * wt, 32), :], osem[buf]
        )

    def complete(j, buf):
        wait_in(start + j, buf)
        pl.when(j >= 4)(lambda: out_copy(start + j - 4, buf).wait())
        transpose(buf)
        out_copy(start + j, buf).start()

    start_in(start, 0)
    start_in(start + 1, 1)
    start_in(start + 2, 2)

    @pl.loop(3, TRIPS - 3, step=4)
    def _steady(t):
        for d in range(4):
            start_in(start + t + d, (3 + d) % 4)
            complete(t + d - 3, d % 4)

    start_in(start + 247, 3)
    complete(244, 0)
    start_in(start + 248, 0)
    complete(245, 1)
    start_in(start + 249, 1)
    complete(246, 2)
    complete(247, 3)
    complete(248, 0)
    complete(249, 1)
    out_copy(start + 246, 2).wait()
    out_copy(start + 247, 3).wait()
    out_copy(start + 248, 0).wait()
    out_copy(start + 249, 1).wait()


@functools.partial(
    pl.kernel,
    out_type=jax.ShapeDtypeStruct((HIST, VECDIM // 8, NBT, 8, BBLK), jnp.float32),
    mesh=_mesh,
    scratch_types=[
        pltpu.VMEM((TAIL, VECDIM), jnp.float32),
        pltpu.VMEM((BPW,), jnp.int32),
        pltpu.VMEM((BBLK,), jnp.int32),
        pltpu.VMEM((BBLK,), jnp.int32),
        pltpu.VMEM((BBLK, VECDIM), jnp.float32),
        pltpu.VMEM((BBLK, VECDIM), jnp.float32),
        pltpu.VMEM((VECDIM // 8, 8, BBLK + 1), jnp.float32),
        pltpu.VMEM((VECDIM // 8, 8, BBLK + 1), jnp.float32),
        pltpu.SemaphoreType.DMA,
        pltpu.SemaphoreType.DMA,
        pltpu.SemaphoreType.DMA,
        pltpu.SemaphoreType.DMA,
    ],
    compiler_params=pltpu.CompilerParams(
        use_tc_tiling_on_sc=False,
        needs_layout_passes=False,
        disable_bounds_checks=True,
    ),
)
def _gather_kernel(idx_hbm, table_flat_hbm, tail_hbm, out_hbm, tail_v, idx_v,
                   ih0, ih1, r0, r1, o0, o1, gsem0, gsem1, wsem0, wsem1):
    wid = lax.axis_index("s") * NC + lax.axis_index("c")
    base = wid * BPW
    table_hbm = table_flat_hbm
    LIMIT = 128 * NTILE_BULK

    ih = (ih0, ih1)
    rows = (r0, r1)
    oblk = (o0, o1)
    gsem = (gsem0, gsem1)
    wsem = (wsem0, wsem1)

    iota = jnp.arange(16, dtype=jnp.int32)

    pltpu.sync_copy(idx_hbm.at[pl.ds(base, BPW)], idx_v)
    pltpu.sync_copy(tail_hbm, tail_v)

    def block_params(t):
        bt_i = t // HIST
        h = t - bt_i * HIST
        return bt_i, h

    def repack_and_gather(t, buf):
        bt_i, h = block_params(t)
        pos0 = bt_i * (HIST * BBLK) + h
        for k in range(8):
            vec = plsc.load_gather(idx_v, [iota * HIST + (pos0 + 800 * k)])
            ih[buf][pl.ds(16 * k, 16)] = vec
        return pltpu.async_copy(table_hbm.at[ih[buf]], rows[buf], gsem[buf])

    def wait_gather(buf):
        pltpu.make_async_copy(table_hbm.at[ih[buf]], rows[buf], gsem[buf]).wait()

    def transpose_and_write(t, buf):
        bt_i, h = block_params(t)

        @plsc.parallel_loop(0, BBLK, step=1, unroll=4)
        def _l(l):
            lvec = jnp.full((16,), 0, dtype=jnp.int32) + l
            for c0 in (0, 16):
                vec = rows[buf][l, pl.ds(c0, 16)]
                ct_vec = (c0 + iota) // 8
                s_vec = (c0 + iota) % 8
                plsc.store_scatter(oblk[buf], [ct_vec, s_vec, lvec], vec)

        acc = jnp.zeros((16,), dtype=jnp.int32)
        for m in range(8):
            tl = ih[buf][pl.ds(16 * m, 16)]
            acc = acc + (tl >= LIMIT).astype(jnp.int32)
        any_tail = lax.reduce_max(acc, (0,))

        @pl.when(any_tail > 0)
        def _fixup():
            for m in range(8):
                tl = ih[buf][pl.ds(16 * m, 16)]
                mask = tl >= LIMIT
                tv = jnp.maximum(tl - LIMIT, 0)
                lv = iota + 16 * m
                for c in range(VECDIM):
                    cv = jnp.full((16,), c, dtype=jnp.int32)
                    vals = plsc.load_gather(tail_v, [tv, cv], mask=mask)
                    plsc.store_scatter(
                        oblk[buf],
                        [jnp.full((16,), c // 8, dtype=jnp.int32),
                         jnp.full((16,), c % 8, dtype=jnp.int32), lv],
                        vals, mask=mask,
                    )

        dst = out_hbm.at[h, :, wid * BT_PER_W + bt_i]
        src = oblk[buf].at[:, :, pl.ds(0, BBLK)]
        return pltpu.async_copy(src, dst, wsem[buf])

    def wait_write(t, buf):
        bt_i, h = block_params(t)
        dst = out_hbm.at[h, :, wid * BT_PER_W + bt_i]
        src = oblk[buf].at[:, :, pl.ds(0, BBLK)]
        pltpu.make_async_copy(src, dst, wsem[buf]).wait()

    def half(t, tbuf):
        repack_and_gather(t, tbuf)
        prev = t - 1
        pbuf = 1 - tbuf
        wait_gather(pbuf)
        pl.when(t >= 3)(lambda: wait_write(prev - 2, pbuf))
        transpose_and_write(prev, pbuf)

    repack_and_gather(0, 0)

    @pl.loop(1, NBLOCK - 1, step=2)
    def _steady(t):
        half(t, 1)
        half(t + 1, 0)

    t_last = NBLOCK - 1
    repack_and_gather(t_last, 1)
    wait_gather(0)
    wait_write(t_last - 3, 0)
    transpose_and_write(t_last - 1, 0)
    wait_gather(1)
    wait_write(t_last - 2, 1)
    transpose_and_write(t_last, 1)
    wait_write(t_last - 1, 0)
    wait_write(t_last, 1)


def kernel(x, table):
    idx = x.reshape(B).astype(jnp.int32)
    table_dense = _detile_kernel(table.T).reshape(WORDNUM, VECDIM)
    tail = lax.slice(table, (128 * NTILE_BULK, 0), (WORDNUM, VECDIM))
    out5d = _gather_kernel(idx, table_dense, tail)
    return out5d.transpose(2, 4, 0, 1, 3).reshape(BATCH, HIST, VECDIM)
